# baseline (speedup 1.0000x reference)
import functools

import jax
import jax.numpy as jnp
from jax import lax
from jax.experimental import pallas as pl
from jax.experimental.pallas import tpu as pltpu
from jax.experimental.pallas import tpu_sc as plsc

NODE = 100000
D = 128
B = 16384
H = 20
NEG = 20

NC = 2
NS = 16
NW = NC * NS
BPW = B // NW
CH = 8
NCHUNK = BPW // CH
HHALF = CH * H // 2


def _sc_scores(source, target, h_s_flat, nt, embeddings, delta1d):
    mesh = plsc.VectorSubcoreMesh(
        core_axis_name="c", subcore_axis_name="s",
        num_cores=NC, num_subcores=NS)

    bufset = [
        pltpu.VMEM((2 * CH, D), jnp.float32),
        pltpu.VMEM((CH * H, D), jnp.float32),
        pltpu.VMEM((CH, 32), jnp.float32),
        pltpu.VMEM((CH, 32), jnp.float32),
    ]

    @functools.partial(
        pl.kernel,
        out_type=(jax.ShapeDtypeStruct((B, 32), jnp.float32),
                  jax.ShapeDtypeStruct((B, 32), jnp.float32),
                  jax.ShapeDtypeStruct((B,), jnp.float32)),
        mesh=mesh,
        compiler_params=pltpu.CompilerParams(needs_layout_passes=False),
        scratch_types=[
            pltpu.VMEM((NEG,), jnp.int32),
            pltpu.VMEM((NEG, D), jnp.float32),
            pltpu.VMEM((BPW,), jnp.int32),
            pltpu.VMEM((2 * BPW,), jnp.int32),
            pltpu.VMEM((BPW * H,), jnp.int32),
            pltpu.VMEM((BPW,), jnp.float32),
            *bufset, *bufset, *bufset, *bufset,
            pltpu.SemaphoreType.DMA,
            pltpu.SemaphoreType.DMA,
            pltpu.SemaphoreType.DMA,
            pltpu.SemaphoreType.DMA,
            pltpu.SemaphoreType.DMA,
            pltpu.SemaphoreType.DMA,
            pltpu.SemaphoreType.DMA,
            pltpu.SemaphoreType.DMA,
            pltpu.SemaphoreType.DMA,
        ],
    )
    def k(src_h, tgt_h, hs_h, nt_h, emb_h, dlt_h, a_out, n_out, d_out, *scr):
        nt_idx, neg_rows = scr[0], scr[1]
        srcidx_v, stidx_v, hidx_v, dlt_all = scr[2], scr[3], scr[4], scr[5]
        sets = (scr[6:10], scr[10:14], scr[14:18], scr[18:22])
        sems = (scr[22], scr[23], scr[24], scr[25])
        sems_o = (scr[26], scr[27], scr[28], scr[29])
        sem_pre = scr[30]
        wid = lax.axis_index("s") * NC + lax.axis_index("c")
        base = wid * BPW
        pltpu.sync_copy(nt_h, nt_idx)
        pltpu.async_copy(emb_h.at[nt_idx], neg_rows, sem_pre).wait()
        pltpu.sync_copy(src_h.at[pl.ds(base, BPW)], srcidx_v)
        pltpu.sync_copy(tgt_h.at[pl.ds(2 * base, 2 * BPW)], stidx_v)
        pltpu.sync_copy(hs_h.at[pl.ds(base * H, BPW * H)], hidx_v)
        for q in range(4):
            pltpu.async_copy(
                dlt_h.at[srcidx_v.at[pl.ds(q * 128, 128)]],
                dlt_all.at[pl.ds(q * 128, 128)], sem_pre)
        lanes = lax.iota(jnp.int32, 16)

        def dma_list(c, bs):
            st_rows, h_rows = sets[bs][0], sets[bs][1]
            loc = c * CH
            return [
                (emb_h.at[stidx_v.at[pl.ds(2 * loc, 2 * CH)]], st_rows),
                (emb_h.at[hidx_v.at[pl.ds(loc * H, HHALF)]],
                 h_rows.at[pl.ds(0, HHALF)]),
                (emb_h.at[hidx_v.at[pl.ds(loc * H + HHALF, HHALF)]],
                 h_rows.at[pl.ds(HHALF, HHALF)]),
            ]

        def issue(c, bs):
            for src, dst in dma_list(c, bs):
                pltpu.async_copy(src, dst, sems[bs])

        def drain(c, bs):
            for src, dst in dma_list(c, bs):
                pltpu.make_async_copy(src, dst, sems[bs]).wait()

        def out_drain(c, bs):
            a_v, n_v = sets[bs][2], sets[bs][3]
            off = base + c * CH
            pltpu.make_async_copy(a_v, a_out.at[pl.ds(off, CH)],
                                  sems_o[bs]).wait()
            pltpu.make_async_copy(n_v, n_out.at[pl.ds(off, CH)],
                                  sems_o[bs]).wait()

        def compute(c, bs):
            st_rows, h_rows, a_v, n_v = sets[bs]
            off = base + c * CH

            @plsc.parallel_loop(0, CH)
            def elem_body(e):
                svec = [st_rows[e, pl.ds(16 * kk, 16)] for kk in range(8)]

                def dist(row_ref, ridx):
                    dd = svec[0] - row_ref[ridx, pl.ds(0, 16)]
                    acc = dd * dd
                    for kk in range(1, 8):
                        dd = svec[kk] - row_ref[ridx, pl.ds(16 * kk, 16)]
                        acc = acc + dd * dd
                    return jnp.sum(acc)

                zz = jnp.zeros((16,), jnp.float32)

                def hbody(h, car):
                    a0, a1, n0, n1 = car
                    dv = dist(h_rows, e * H + h)
                    nv = dist(neg_rows, h)
                    lo = (lanes == h) & (h < 16)
                    hi = (lanes == (h - 16)) & (h >= 16)
                    return (jnp.where(lo, dv, a0), jnp.where(hi, dv, a1),
                            jnp.where(lo, nv, n0), jnp.where(hi, nv, n1))

                a0, a1, n0, n1 = lax.fori_loop(0, H, hbody,
                                               (zz, zz, zz, zz))
                pmu = dist(st_rows, CH + e)
                a1 = jnp.where(lanes == (H - 16), pmu, a1)
                a_v[e, pl.ds(0, 16)] = a0
                a_v[e, pl.ds(16, 16)] = a1
                n_v[e, pl.ds(0, 16)] = n0
                n_v[e, pl.ds(16, 16)] = n1
            pltpu.async_copy(a_v, a_out.at[pl.ds(off, CH)], sems_o[bs])
            pltpu.async_copy(n_v, n_out.at[pl.ds(off, CH)], sems_o[bs])

        NSET = 4
        for p in range(NSET):
            issue(p, p)

        def quad_body(g, carry):
            for b4 in range(NSET):
                c = NSET * g + b4
                drain(c, b4)

                @pl.when(c >= NSET)
                def _():
                    out_drain(c - NSET, b4)

                compute(c, b4)
                nxt = c + NSET

                @pl.when(nxt < NCHUNK)
                def _():
                    issue(nxt, b4)
            return carry

        lax.fori_loop(0, NCHUNK // NSET, quad_body, 0)
        for b4 in range(NSET):
            out_drain(NCHUNK - NSET + b4, b4)
        for q in range(4):
            pltpu.make_async_copy(
                dlt_h.at[srcidx_v.at[pl.ds(q * 128, 128)]],
                dlt_all.at[pl.ds(q * 128, 128)], sem_pre).wait()
        pltpu.sync_copy(dlt_all, d_out.at[pl.ds(base, BPW)])

    return k(source, target, h_s_flat, nt, embeddings, delta1d)


def _tc_finish(a_pk, n_pk, dlt2, times2, h_s_times, h_s_mask):
    BLK = 2048

    def body(a_ref, n_ref, d_ref, t_ref, ht_ref, hm_ref, o_ref):
        a_full = a_ref[...]
        alpha = -a_full[:, :H]
        pmu = -a_full[:, H:H + 1]
        dlt = d_ref[...]
        nmu = -n_ref[...][:, :H]
        m = jnp.max(alpha, axis=1, keepdims=True)
        ex = jnp.exp(alpha - m)
        attn = ex / jnp.sum(ex, axis=1, keepdims=True)
        d_time = t_ref[...] - ht_ref[...]
        dec = jnp.exp(-dlt * d_time)
        p_lam = pmu + jnp.sum(attn * alpha * dec * hm_ref[...],
                              axis=1, keepdims=True)
        n_lam = jnp.sum(attn * nmu * dec, axis=1, keepdims=True)
        o_ref[...] = -jax.nn.log_sigmoid(p_lam) - jax.nn.log_sigmoid(-n_lam)

    grid = (B // BLK,)
    return pl.pallas_call(
        body,
        grid=grid,
        in_specs=[pl.BlockSpec((BLK, 32), lambda i: (i, 0)),
                  pl.BlockSpec((BLK, 32), lambda i: (i, 0)),
                  pl.BlockSpec((BLK, 1), lambda i: (i, 0)),
                  pl.BlockSpec((BLK, 1), lambda i: (i, 0)),
                  pl.BlockSpec((BLK, H), lambda i: (i, 0)),
                  pl.BlockSpec((BLK, H), lambda i: (i, 0))],
        out_specs=pl.BlockSpec((BLK, 1), lambda i: (i, 0)),
        out_shape=jax.ShapeDtypeStruct((B, 1), jnp.float32),
    )(a_pk, n_pk, dlt2, times2, h_s_times, h_s_mask)


def kernel(source, target, times, h_s, h_s_times, h_s_mask, nt,
           embeddings, delta_table):
    h_s_flat = h_s.reshape(-1).astype(jnp.int32)
    src32 = source.astype(jnp.int32)
    st_comb = jnp.concatenate(
        [src32.reshape(NW, NCHUNK, CH),
         target.astype(jnp.int32).reshape(NW, NCHUNK, CH)],
        axis=2).reshape(-1)
    a_pk, n_pk, dlt = _sc_scores(src32, st_comb,
                                 h_s_flat, nt.astype(jnp.int32),
                                 embeddings, delta_table.reshape(-1))
    out2 = _tc_finish(a_pk, n_pk, dlt[:, None], times[:, None],
                      h_s_times, h_s_mask)
    return out2.reshape(B)


# speedup vs baseline: 1.1291x; 1.1291x over previous
import functools

import jax
import jax.numpy as jnp
from jax import lax
from jax.experimental import pallas as pl
from jax.experimental.pallas import tpu as pltpu
from jax.experimental.pallas import tpu_sc as plsc

NODE = 100000
D = 128
B = 16384
H = 20
NEG = 20

NC = 2
NS = 16
NW = NC * NS
BPW = B // NW
CH = 8
NCHUNK = BPW // CH
HHALF = CH * H // 2


def _sc_scores(source, target, h_s_flat, nt, embeddings, delta1d):
    mesh = plsc.VectorSubcoreMesh(
        core_axis_name="c", subcore_axis_name="s",
        num_cores=NC, num_subcores=NS)

    bufset = [
        pltpu.VMEM((2 * CH, D), jnp.float32),
        pltpu.VMEM((CH * H, D), jnp.float32),
    ]

    @functools.partial(
        pl.kernel,
        out_type=(jax.ShapeDtypeStruct((B, 32), jnp.float32),
                  jax.ShapeDtypeStruct((B, 32), jnp.float32),
                  jax.ShapeDtypeStruct((B,), jnp.float32)),
        mesh=mesh,
        compiler_params=pltpu.CompilerParams(needs_layout_passes=False),
        scratch_types=[
            pltpu.VMEM((NEG,), jnp.int32),
            pltpu.VMEM((NEG, D), jnp.float32),
            pltpu.VMEM((BPW,), jnp.int32),
            pltpu.VMEM((2 * BPW,), jnp.int32),
            pltpu.VMEM((BPW * H,), jnp.int32),
            pltpu.VMEM((BPW,), jnp.float32),
            *bufset, *bufset,
            pltpu.VMEM((CH, 32), jnp.float32),
            pltpu.VMEM((CH, 32), jnp.float32),
            pltpu.VMEM((CH, 32), jnp.float32),
            pltpu.VMEM((CH, 32), jnp.float32),
            pltpu.SemaphoreType.DMA,
            pltpu.SemaphoreType.DMA,
            pltpu.SemaphoreType.DMA,
            pltpu.SemaphoreType.DMA,
            pltpu.SemaphoreType.DMA,
        ],
    )
    def k(src_h, tgt_h, hs_h, nt_h, emb_h, dlt_h, a_out, n_out, d_out, *scr):
        nt_idx, neg_rows = scr[0], scr[1]
        srcidx_v, stidx_v, hidx_v, dlt_all = scr[2], scr[3], scr[4], scr[5]
        sets = (scr[6:8], scr[8:10])
        outv = ((scr[10], scr[11]), (scr[12], scr[13]))
        sems = (scr[14], scr[15])
        sems_o = (scr[16], scr[17])
        sem_pre = scr[18]
        wid = lax.axis_index("s") * NC + lax.axis_index("c")
        base = wid * BPW
        pltpu.sync_copy(nt_h, nt_idx)
        pltpu.async_copy(emb_h.at[nt_idx], neg_rows, sem_pre).wait()
        pltpu.sync_copy(src_h.at[pl.ds(base, BPW)], srcidx_v)
        pltpu.sync_copy(tgt_h.at[pl.ds(2 * base, 2 * BPW)], stidx_v)
        pltpu.sync_copy(hs_h.at[pl.ds(base * H, BPW * H)], hidx_v)
        for q in range(4):
            pltpu.async_copy(
                dlt_h.at[srcidx_v.at[pl.ds(q * 128, 128)]],
                dlt_all.at[pl.ds(q * 128, 128)], sem_pre)
        lanes = lax.iota(jnp.int32, 16)

        def dma_list(c, bs):
            st_rows, h_rows = sets[bs]
            loc = c * CH
            return [
                (emb_h.at[stidx_v.at[pl.ds(2 * loc, 2 * CH)]], st_rows),
                (emb_h.at[hidx_v.at[pl.ds(loc * H, HHALF)]],
                 h_rows.at[pl.ds(0, HHALF)]),
                (emb_h.at[hidx_v.at[pl.ds(loc * H + HHALF, HHALF)]],
                 h_rows.at[pl.ds(HHALF, HHALF)]),
            ]

        def issue(c, bs):
            for src, dst in dma_list(c, bs):
                pltpu.async_copy(src, dst, sems[bs])

        def drain(c, bs):
            for src, dst in dma_list(c, bs):
                pltpu.make_async_copy(src, dst, sems[bs]).wait()

        def out_drain(c, bs):
            a_v, n_v = outv[bs]
            off = base + c * CH
            pltpu.make_async_copy(a_v, a_out.at[pl.ds(off, CH)],
                                  sems_o[bs]).wait()
            pltpu.make_async_copy(n_v, n_out.at[pl.ds(off, CH)],
                                  sems_o[bs]).wait()

        def compute(c, bs):
            st_rows, h_rows = sets[bs]
            a_v, n_v = outv[bs]
            off = base + c * CH

            @plsc.parallel_loop(0, CH)
            def elem_body(e):
                svec = [st_rows[e, pl.ds(16 * kk, 16)] for kk in range(8)]

                def dist(row_ref, ridx):
                    dd = svec[0] - row_ref[ridx, pl.ds(0, 16)]
                    acc = dd * dd
                    for kk in range(1, 8):
                        dd = svec[kk] - row_ref[ridx, pl.ds(16 * kk, 16)]
                        acc = acc + dd * dd
                    return jnp.sum(acc)

                a0 = jnp.zeros((16,), jnp.float32)
                a1 = jnp.zeros((16,), jnp.float32)
                n0 = jnp.zeros((16,), jnp.float32)
                n1 = jnp.zeros((16,), jnp.float32)
                for h in range(H):
                    dv = dist(h_rows, e * H + h)
                    nv = dist(neg_rows, h)
                    if h < 16:
                        a0 = jnp.where(lanes == h, dv, a0)
                        n0 = jnp.where(lanes == h, nv, n0)
                    else:
                        a1 = jnp.where(lanes == (h - 16), dv, a1)
                        n1 = jnp.where(lanes == (h - 16), nv, n1)
                pmu = dist(st_rows, CH + e)
                a1 = jnp.where(lanes == (H - 16), pmu, a1)
                a_v[e, pl.ds(0, 16)] = a0
                a_v[e, pl.ds(16, 16)] = a1
                n_v[e, pl.ds(0, 16)] = n0
                n_v[e, pl.ds(16, 16)] = n1
            pltpu.async_copy(a_v, a_out.at[pl.ds(off, CH)], sems_o[bs])
            pltpu.async_copy(n_v, n_out.at[pl.ds(off, CH)], sems_o[bs])

        issue(0, 0)
        issue(1, 1)

        def pair_body(g, carry):
            for b2 in range(2):
                c = 2 * g + b2
                drain(c, b2)

                @pl.when(c >= 2)
                def _():
                    out_drain(c - 2, b2)

                compute(c, b2)
                nxt = c + 2

                @pl.when(nxt < NCHUNK)
                def _():
                    issue(nxt, b2)
            return carry

        lax.fori_loop(0, NCHUNK // 2, pair_body, 0)
        out_drain(NCHUNK - 2, 0)
        out_drain(NCHUNK - 1, 1)
        for q in range(4):
            pltpu.make_async_copy(
                dlt_h.at[srcidx_v.at[pl.ds(q * 128, 128)]],
                dlt_all.at[pl.ds(q * 128, 128)], sem_pre).wait()
        pltpu.sync_copy(dlt_all, d_out.at[pl.ds(base, BPW)])

    return k(source, target, h_s_flat, nt, embeddings, delta1d)


def _tc_finish(a_pk, n_pk, dlt2, times2, h_s_times, h_s_mask):
    BLK = 2048

    def body(a_ref, n_ref, d_ref, t_ref, ht_ref, hm_ref, o_ref):
        a_full = a_ref[...]
        alpha = -a_full[:, :H]
        pmu = -a_full[:, H:H + 1]
        dlt = d_ref[...]
        nmu = -n_ref[...][:, :H]
        m = jnp.max(alpha, axis=1, keepdims=True)
        ex = jnp.exp(alpha - m)
        attn = ex / jnp.sum(ex, axis=1, keepdims=True)
        d_time = t_ref[...] - ht_ref[...]
        dec = jnp.exp(-dlt * d_time)
        p_lam = pmu + jnp.sum(attn * alpha * dec * hm_ref[...],
                              axis=1, keepdims=True)
        n_lam = jnp.sum(attn * nmu * dec, axis=1, keepdims=True)
        o_ref[...] = -jax.nn.log_sigmoid(p_lam) - jax.nn.log_sigmoid(-n_lam)

    grid = (B // BLK,)
    return pl.pallas_call(
        body,
        grid=grid,
        in_specs=[pl.BlockSpec((BLK, 32), lambda i: (i, 0)),
                  pl.BlockSpec((BLK, 32), lambda i: (i, 0)),
                  pl.BlockSpec((BLK, 1), lambda i: (i, 0)),
                  pl.BlockSpec((BLK, 1), lambda i: (i, 0)),
                  pl.BlockSpec((BLK, H), lambda i: (i, 0)),
                  pl.BlockSpec((BLK, H), lambda i: (i, 0))],
        out_specs=pl.BlockSpec((BLK, 1), lambda i: (i, 0)),
        out_shape=jax.ShapeDtypeStruct((B, 1), jnp.float32),
    )(a_pk, n_pk, dlt2, times2, h_s_times, h_s_mask)


def kernel(source, target, times, h_s, h_s_times, h_s_mask, nt,
           embeddings, delta_table):
    h_s_flat = h_s.reshape(-1).astype(jnp.int32)
    src32 = source.astype(jnp.int32)
    st_comb = jnp.concatenate(
        [src32.reshape(NW, NCHUNK, CH),
         target.astype(jnp.int32).reshape(NW, NCHUNK, CH)],
        axis=2).reshape(-1)
    a_pk, n_pk, dlt = _sc_scores(src32, st_comb,
                                 h_s_flat, nt.astype(jnp.int32),
                                 embeddings, delta_table.reshape(-1))
    out2 = _tc_finish(a_pk, n_pk, dlt[:, None], times[:, None],
                      h_s_times, h_s_mask)
    return out2.reshape(B)
